# baseline (speedup 1.0000x reference)
import functools

import jax
import jax.numpy as jnp
from jax import lax
from jax.experimental import pallas as pl
from jax.experimental.pallas import tpu as pltpu
from jax.experimental.pallas import tpu_sc as plsc

NA = 10000
NP = 10000
NE = 320000
D = 128

NC = 2
NS = 16
NW = NC * NS
L = 16

NROW = 10240
JROW = 10000
TE = NROW
EPAD = NW * TE
CHUNK = 128
NCHUNK = TE // CHUNK
RPT = NROW // NS
PCH = 64
PPT = NROW // NW

_mesh = plsc.VectorSubcoreMesh(
    core_axis_name="c", subcore_axis_name="s", num_cores=NC, num_subcores=NS)

_f32 = jnp.float32
_i32 = jnp.int32


COLW = 8
BE = 1024
NBK = (EPAD // NC) // BE
NBH = NBK // 2


def _relayout_body(g_hbm, h_hbm, colout, ig, r0, r1, semw0, semw1):
  cid = lax.axis_index("c")
  sid = lax.axis_index("s")
  base1 = (cid * NS + sid) * TE

  def do_chunk(i, rbuf, semw):
    off = base1 + i * CHUNK
    pltpu.sync_copy(g_hbm.at[pl.ds(off, CHUNK)], ig)
    pltpu.sync_copy(h_hbm.at[ig], rbuf)
    for t in range(NS):
      pltpu.async_copy(rbuf.at[:, pl.ds(COLW * t, COLW)],
                       colout.at[t, pl.ds(off, CHUNK)], semw)

  def drain(rbuf, semw):
    for t in range(NS):
      pltpu.make_async_copy(rbuf.at[:, pl.ds(COLW * t, COLW)],
                            colout.at[0, pl.ds(base1, CHUNK)], semw).wait()

  do_chunk(0, r0, semw0)
  do_chunk(1, r1, semw1)

  @pl.loop(1, NCHUNK // 2)
  def _(p):
    drain(r0, semw0)
    do_chunk(2 * p, r0, semw0)
    drain(r1, semw1)
    do_chunk(2 * p + 1, r1, semw1)

  drain(r0, semw0)
  drain(r1, semw1)


def _relayout(gidx, table):
  return pl.kernel(
      _relayout_body,
      out_type=[jax.ShapeDtypeStruct((NS, EPAD, COLW), _f32)],
      mesh=_mesh,
      compiler_params=pltpu.CompilerParams(needs_layout_passes=False, use_tc_tiling_on_sc=False),
      scratch_types=[
          pltpu.VMEM((CHUNK,), _i32),
          pltpu.VMEM((CHUNK, D), _f32),
          pltpu.VMEM((CHUNK, D), _f32),
          pltpu.SemaphoreType.DMA,
          pltpu.SemaphoreType.DMA,
      ],
  )(gidx, table)[0]


def _make_accum(weighted):
  S2 = 2 if weighted else 1
  ISZ = 8 * S2

  def body(*refs):
    if weighted:
      pk_hbm, col_hbm, out4 = refs[:3]
      iA, iB, cA, cB, acc, siA, siB, scA, scB = refs[3:]
      outcnt = cnt = None
    else:
      pk_hbm, col_hbm, out4, outcnt = refs[:4]
      iA, iB, cA, cB, acc, cnt, siA, siB, scA, scB = refs[4:]
    cid = lax.axis_index("c")
    sid = lax.axis_index("s")
    zero16 = jnp.zeros((L,), _f32)
    ones16 = jnp.full((L,), 1.0, _f32)

    @pl.loop(0, NROW * COLW // L)
    def _(i):
      acc[pl.ds(i * L, L)] = zero16

    if not weighted:
      @pl.loop(0, NROW // L)
      def _(i):
        cnt[pl.ds(i * L, L)] = zero16

    ebase = cid * (EPAD // NC)
    pkbase = (ebase // CHUNK) * S2

    def idx_load(ibuf, sem, b):
      pltpu.async_copy(pk_hbm.at[pl.ds(pkbase + b * ISZ, ISZ)], ibuf, sem)

    def idx_wait(ibuf, sem):
      pltpu.make_async_copy(pk_hbm.at[pl.ds(pkbase, ISZ)], ibuf, sem).wait()

    def col_load(cbuf, sem, b):
      pltpu.async_copy(col_hbm.at[sid, pl.ds(ebase + b * BE, BE)], cbuf, sem)

    def col_wait(cbuf, sem):
      pltpu.make_async_copy(col_hbm.at[sid, pl.ds(ebase, BE)], cbuf, sem).wait()

    def process(ibuf, cbuf):
      for r in range(8):
        for j in range(CHUNK // L):
          sl = pl.ds(j * L, L)
          d16 = ibuf[S2 * r, sl]
          fbase = d16 * COLW
          row16 = lax.iota(_i32, L) + (r * CHUNK + j * L)
          if weighted:
            w16 = plsc.bitcast(ibuf[S2 * r + 1, sl], _f32)
          for c in range(COLW):
            csp = jnp.full((L,), c, _i32)
            val = plsc.load_gather(cbuf, [row16, csp])
            if weighted:
              val = val * w16
            plsc.addupdate_scatter(acc, [fbase + c], val)
          if not weighted:
            @pl.when(sid == 0)
            def _():
              plsc.addupdate_scatter(cnt, [d16], ones16)

    idx_load(iA, siA, 0)
    col_load(cA, scA, 0)

    @pl.loop(0, NBH)
    def _(p):
      b0 = 2 * p
      idx_load(iB, siB, b0 + 1)
      col_load(cB, scB, b0 + 1)
      idx_wait(iA, siA)
      col_wait(cA, scA)
      process(iA, cA)
      nxt = jnp.minimum(b0 + 2, NBK - 1)
      idx_load(iA, siA, nxt)
      col_load(cA, scA, nxt)
      idx_wait(iB, siB)
      col_wait(cB, scB)
      process(iB, cB)

    idx_wait(iA, siA)
    col_wait(cA, scA)

    pltpu.sync_copy(acc, out4.at[cid, sid])
    if not weighted:
      @pl.when(sid == 0)
      def _():
        pltpu.sync_copy(cnt, outcnt.at[cid])

  outs = [jax.ShapeDtypeStruct((NC, NS, NROW * COLW), _f32)]
  scratch = [
      pltpu.VMEM((ISZ, CHUNK), _i32),
      pltpu.VMEM((ISZ, CHUNK), _i32),
      pltpu.VMEM((BE, COLW), _f32),
      pltpu.VMEM((BE, COLW), _f32),
      pltpu.VMEM((NROW * COLW,), _f32),
  ]
  if not weighted:
    outs.append(jax.ShapeDtypeStruct((NC, NROW), _f32))
    scratch.append(pltpu.VMEM((NROW,), _f32))
  scratch += [pltpu.SemaphoreType.DMA] * 4

  def call(pk, col):
    return pl.kernel(
        body,
        out_type=outs,
        mesh=_mesh,
        compiler_params=pltpu.CompilerParams(needs_layout_passes=False, use_tc_tiling_on_sc=False),
        scratch_types=scratch,
    )(pk, col)

  return call


_accum_sage = _make_accum(weighted=False)
_accum_w = _make_accum(weighted=True)


def _pack2(a, b):
  bi = b if b.dtype == _i32 else lax.bitcast_convert_type(b, _i32)
  return jnp.stack(
      [a.reshape(-1, CHUNK), bi.reshape(-1, CHUNK)], axis=1).reshape(-1, CHUNK)


def _unT(o4):
  return o4.reshape(NC, NS, NROW, COLW).transpose(0, 2, 1, 3).reshape(
      NC, NROW, D)


def _gatw_body(src_hbm, pap_hbm, p2a_hbm, as_hbm, ad_hbm, perm_hbm,
               out_wp, out_wn, out_gn, out_dd, out_denp, out_denn,
               sbuf, pbuf, wpb, wnb, gnb, ddb, p2a_v, as_v, ad_v, perm_v,
               denp_v, denn_v):
  cid = lax.axis_index("c")
  sid = lax.axis_index("s")
  wid = sid * NC + cid
  zero16 = jnp.zeros((L,), _f32)

  pltpu.sync_copy(p2a_hbm, p2a_v)
  pltpu.sync_copy(as_hbm, as_v)
  pltpu.sync_copy(ad_hbm, ad_v)
  pltpu.sync_copy(perm_hbm, perm_v)

  @pl.loop(0, NROW // L)
  def _(i):
    denp_v[pl.ds(i * L, L)] = zero16
    denn_v[pl.ds(i * L, L)] = zero16

  base = wid * TE

  @pl.loop(0, NCHUNK)
  def _(i):
    off = base + i * CHUNK
    pltpu.sync_copy(src_hbm.at[pl.ds(off, CHUNK)], sbuf)
    pltpu.sync_copy(pap_hbm.at[pl.ds(off, CHUNK)], pbuf)
    for j in range(CHUNK // L):
      sl = pl.ds(j * L, L)
      s16 = sbuf[sl]
      p16 = pbuf[sl]
      mapped = plsc.load_gather(p2a_v, [p16])
      mask = mapped >= 0
      d0 = jnp.where(mask, mapped, 0)
      dsc = jnp.where(mask, mapped, JROW)

      def edge_w(si, di):
        e = plsc.load_gather(as_v, [si]) + plsc.load_gather(ad_v, [di])
        e = jnp.where(e >= 0.0, e, e * 0.2)
        return jnp.where(mask, jnp.exp(e), 0.0)

      wp = edge_w(s16, d0)
      sgn = plsc.load_gather(perm_v, [s16])
      dgn = plsc.load_gather(perm_v, [d0])
      wn = edge_w(sgn, dgn)
      plsc.addupdate_scatter(denp_v, [dsc], wp, mask=mask)
      plsc.addupdate_scatter(denn_v, [dsc], wn, mask=mask)
      wpb[sl] = wp
      wnb[sl] = wn
      gnb[sl] = sgn
      ddb[sl] = dsc
    pltpu.sync_copy(wpb, out_wp.at[pl.ds(off, CHUNK)])
    pltpu.sync_copy(wnb, out_wn.at[pl.ds(off, CHUNK)])
    pltpu.sync_copy(gnb, out_gn.at[pl.ds(off, CHUNK)])
    pltpu.sync_copy(ddb, out_dd.at[pl.ds(off, CHUNK)])

  pltpu.sync_copy(denp_v, out_denp.at[wid])
  pltpu.sync_copy(denn_v, out_denn.at[wid])


def _gat_weights(src_ap, pap_ap, p2a_pad, a_s, a_d, perm_tab):
  return pl.kernel(
      _gatw_body,
      out_type=[jax.ShapeDtypeStruct((EPAD,), _f32),
                jax.ShapeDtypeStruct((EPAD,), _f32),
                jax.ShapeDtypeStruct((EPAD,), _i32),
                jax.ShapeDtypeStruct((EPAD,), _i32),
                jax.ShapeDtypeStruct((NW, NROW), _f32),
                jax.ShapeDtypeStruct((NW, NROW), _f32)],
      mesh=_mesh,
      compiler_params=pltpu.CompilerParams(needs_layout_passes=False, use_tc_tiling_on_sc=False),
      scratch_types=[
          pltpu.VMEM((CHUNK,), _i32),
          pltpu.VMEM((CHUNK,), _i32),
          pltpu.VMEM((CHUNK,), _f32),
          pltpu.VMEM((CHUNK,), _f32),
          pltpu.VMEM((CHUNK,), _i32),
          pltpu.VMEM((CHUNK,), _i32),
          pltpu.VMEM((NROW,), _i32),
          pltpu.VMEM((NA,), _f32),
          pltpu.VMEM((NA,), _f32),
          pltpu.VMEM((NA,), _i32),
          pltpu.VMEM((NROW,), _f32),
          pltpu.VMEM((NROW,), _f32),
      ],
  )(src_ap, pap_ap, p2a_pad, a_s, a_d, perm_tab)


def _perm_body(h_hbm, ws_hbm, perm_hbm, out_hp, out_wsp,
               ibuf, rows, wtab, wout):
  cid = lax.axis_index("c")
  sid = lax.axis_index("s")
  wid = sid * NC + cid
  pltpu.sync_copy(ws_hbm, wtab)
  base = wid * PPT

  @pl.loop(0, PPT // PCH)
  def _(k):
    off = base + k * PCH
    pltpu.sync_copy(perm_hbm.at[pl.ds(off, PCH)], ibuf)
    pltpu.sync_copy(h_hbm.at[ibuf], rows)
    pltpu.sync_copy(rows, out_hp.at[pl.ds(off, PCH)])
    for g in range(PCH // L):
      sl = pl.ds(g * L, L)
      wout[sl] = plsc.load_gather(wtab, [ibuf[sl]])
    pltpu.sync_copy(wout, out_wsp.at[pl.ds(off, PCH)])


def _permute(h, ws_flat, perm_pad):
  return pl.kernel(
      _perm_body,
      out_type=[jax.ShapeDtypeStruct((NROW, D), _f32),
                jax.ShapeDtypeStruct((NROW,), _f32)],
      mesh=_mesh,
      compiler_params=pltpu.CompilerParams(needs_layout_passes=False, use_tc_tiling_on_sc=False),
      scratch_types=[
          pltpu.VMEM((PCH,), _i32),
          pltpu.VMEM((PCH, D), _f32),
          pltpu.VMEM((NA,), _f32),
          pltpu.VMEM((PCH,), _f32),
      ],
  )(h, ws_flat, perm_pad)


_BLK = 2048
_GRID = NROW // _BLK


def _tc1_body(sum_ref, cnt_ref, xa_ref, wl_ref, bl_ref, wr_ref, wg_ref,
              as_ref, ad_ref, h_ref, asv_ref, adv_ref, ws_ref):
  dn = (((1,), (1,)), ((), ()))
  s = sum_ref[0] + sum_ref[1]
  c = jnp.sum(cnt_ref[...], axis=0)
  mean = s / jnp.maximum(c, 1.0)[:, None]
  ha = lax.dot_general(mean, wl_ref[...], dn, preferred_element_type=_f32)
  ha = ha + bl_ref[...]
  ha = ha + lax.dot_general(xa_ref[...], wr_ref[...], dn,
                            preferred_element_type=_f32)
  ha = jnp.where(ha >= 0.0, ha, 0.01 * ha)
  h = lax.dot_general(ha, wg_ref[...], dn, preferred_element_type=_f32)
  h_ref[...] = h
  dv = (((1,), (0,)), ((), ()))
  a_s = lax.dot_general(h, as_ref[...], dv, preferred_element_type=_f32)
  a_d = lax.dot_general(h, ad_ref[...], dv, preferred_element_type=_f32)
  asv_ref[...] = a_s
  adv_ref[...] = a_d
  e = a_s + a_d
  e = jnp.where(e >= 0.0, e, 0.2 * e)
  ws_ref[...] = jnp.exp(e)


def _tc1(sum2, cnt32, xa_pad, Wl, bl, Wr, Wg, att_s, att_d):
  col = pl.BlockSpec((_BLK, 1), lambda i: (i, 0))
  full = pl.BlockSpec((D, D), lambda i: (0, 0))
  return pl.pallas_call(
      _tc1_body,
      grid=(_GRID,),
      in_specs=[pl.BlockSpec((NC, _BLK, D), lambda i: (0, i, 0)),
                pl.BlockSpec((NC, _BLK), lambda i: (0, i)),
                pl.BlockSpec((_BLK, D), lambda i: (i, 0)),
                full,
                pl.BlockSpec((1, D), lambda i: (0, 0)),
                full, full,
                pl.BlockSpec((D, 1), lambda i: (0, 0)),
                pl.BlockSpec((D, 1), lambda i: (0, 0))],
      out_specs=[pl.BlockSpec((_BLK, D), lambda i: (i, 0)), col, col, col],
      out_shape=[jax.ShapeDtypeStruct((NROW, D), _f32),
                 jax.ShapeDtypeStruct((NROW, 1), _f32),
                 jax.ShapeDtypeStruct((NROW, 1), _f32),
                 jax.ShapeDtypeStruct((NROW, 1), _f32)],
  )(sum2, cnt32, xa_pad, Wl, bl, Wr, Wg, att_s, att_d)


def _tc2_body(np_ref, dp_ref, nn_ref, dn_ref, h_ref, ws_ref, hp_ref, wsp_ref,
              bg_ref, pa_ref, pos_ref, neg_ref, sum_ref):
  i = pl.program_id(0)
  a = pa_ref[0, 0]

  def fin(nref, dref, hv, wv):
    num = nref[0] + nref[1] + wv * hv
    den = jnp.sum(dref[...], axis=0)[:, None] + wv
    o = num / den + bg_ref[...]
    return jnp.where(o >= 0.0, o, a * o)

  pos = fin(np_ref, dp_ref, h_ref[...], ws_ref[...])
  neg = fin(nn_ref, dn_ref, hp_ref[...], wsp_ref[...])
  pos_ref[...] = pos
  neg_ref[...] = neg
  rid = lax.broadcasted_iota(_i32, (_BLK, 1), 0) + i * _BLK
  part = jnp.sum(jnp.where(rid < NA, pos, 0.0), axis=0, keepdims=True)

  @pl.when(i == 0)
  def _():
    sum_ref[...] = jnp.zeros_like(sum_ref)

  sum_ref[...] += part

  @pl.when(i == _GRID - 1)
  def _():
    sum_ref[...] = sum_ref[...] * (1.0 / NA)


def _tc2(nump, denp, numn, denn, h, ws, hp, wsp, bg, pa):
  col = pl.BlockSpec((_BLK, 1), lambda i: (i, 0))
  mat = pl.BlockSpec((_BLK, D), lambda i: (i, 0))
  return pl.pallas_call(
      _tc2_body,
      grid=(_GRID,),
      in_specs=[pl.BlockSpec((NC, _BLK, D), lambda i: (0, i, 0)),
                pl.BlockSpec((NW, _BLK), lambda i: (0, i)),
                pl.BlockSpec((NC, _BLK, D), lambda i: (0, i, 0)),
                pl.BlockSpec((NW, _BLK), lambda i: (0, i)),
                mat, col, mat, col,
                pl.BlockSpec((1, D), lambda i: (0, 0)),
                pl.BlockSpec((1, 1), lambda i: (0, 0))],
      out_specs=[mat, mat, pl.BlockSpec((1, D), lambda i: (0, 0))],
      out_shape=[jax.ShapeDtypeStruct((NROW, D), _f32),
                 jax.ShapeDtypeStruct((NROW, D), _f32),
                 jax.ShapeDtypeStruct((1, D), _f32)],
  )(nump, denp, numn, denn, h, ws, hp, wsp, bg, pa)


@jax.jit
def kernel(x_author, x_paper, W_l_ap, b_l_ap, W_r_ap, W_l_pa, b_l_pa, W_r_pa,
           W_gat, att_src, att_dst, b_gat, prelu_a, edge_index_ap,
           edge_index_pa, perm):
  src_pa = edge_index_pa[0].astype(_i32)
  dst_pa = edge_index_pa[1].astype(_i32)
  src_ap = edge_index_ap[0].astype(_i32)
  pap_ap = edge_index_ap[1].astype(_i32)
  permc = perm.astype(_i32)

  npad = EPAD - NE
  src_pa_p = jnp.concatenate([src_pa, jnp.zeros((npad,), _i32)])
  dst_pa_p = jnp.concatenate([dst_pa, jnp.full((npad,), JROW, _i32)])
  src_ap_p = jnp.concatenate([src_ap, jnp.zeros((npad,), _i32)])
  pap_ap_p = jnp.concatenate([pap_ap, jnp.full((npad,), JROW, _i32)])

  p2a = jnp.full((NP,), -1, _i32).at[src_pa].set(dst_pa)
  p2a_pad = jnp.concatenate([p2a, jnp.full((NROW - NP,), -1, _i32)])
  perm_pad = jnp.concatenate([permc, jnp.zeros((NROW - NA,), _i32)])
  xa_pad = jnp.concatenate([x_author, jnp.zeros((NROW - NA, D), _f32)])

  col_s = _relayout(src_pa_p, x_paper)
  sum4, cnt2 = _accum_sage(dst_pa_p.reshape(-1, CHUNK), col_s)
  sum2 = _unT(sum4)
  h, a_s2, a_d2, ws2 = _tc1(sum2, cnt2, xa_pad, W_l_pa,
                            b_l_pa.reshape(1, D), W_r_pa, W_gat,
                            att_src.reshape(D, 1), att_dst.reshape(D, 1))
  a_s = a_s2[:NA, 0]
  a_d = a_d2[:NA, 0]
  ws_flat = ws2[:NA, 0]
  hp, wsp = _permute(h, ws_flat, perm_pad)
  wp, wn, gn, dd, denp, denn = _gat_weights(src_ap_p, pap_ap_p, p2a_pad,
                                            a_s, a_d, permc)
  col_p = _relayout(src_ap_p, h)
  nump = _unT(_accum_w(_pack2(dd, wp), col_p)[0])
  col_n = _relayout(gn, h)
  numn = _unT(_accum_w(_pack2(dd, wn), col_n)[0])
  pos_f, neg_f, summ = _tc2(nump, denp, numn, denn, h, ws2, hp,
                            wsp.reshape(NROW, 1), b_gat.reshape(1, D),
                            prelu_a.reshape(1, 1))
  return pos_f[:NA], neg_f[:NA], summ.reshape(D)


# speedup vs baseline: 4.0346x; 4.0346x over previous
import functools

import jax
import jax.numpy as jnp
from jax import lax
from jax.experimental import pallas as pl
from jax.experimental.pallas import tpu as pltpu
from jax.experimental.pallas import tpu_sc as plsc

NA = 10000
NP = 10000
NE = 320000
D = 128

NC = 2
NS = 16
NW = NC * NS
L = 16

NROW = 10240
JROW = 10000
TE = NROW
EPAD = NW * TE
CHUNK = 128
NCHUNK = TE // CHUNK
RPT = NROW // NS
PCH = 64
PPT = NROW // NW

_mesh = plsc.VectorSubcoreMesh(
    core_axis_name="c", subcore_axis_name="s", num_cores=NC, num_subcores=NS)

_f32 = jnp.float32
_i32 = jnp.int32


def _sage_body(src_hbm, dst_hbm, xp_hbm, out_sum, out_cnt,
               sidx, didx, rows, cnt, acc):
  cid = lax.axis_index("c")
  sid = lax.axis_index("s")
  wid = sid * NC + cid
  zero16 = jnp.zeros((L,), _f32)

  @pl.loop(0, NROW // L)
  def _(i):
    cnt[pl.ds(i * L, L)] = zero16

  @pl.loop(0, CHUNK)
  def _(r):
    for c in range(D // L):
      rows[r, pl.ds(c * L, L)] = zero16

  for k in range(RPT // CHUNK):
    pltpu.sync_copy(rows, acc.at[pl.ds(sid * RPT + k * CHUNK, CHUNK)])
  plsc.subcore_barrier()

  ones16 = jnp.full((L,), 1.0, _f32)
  base = wid * TE

  @pl.loop(0, NCHUNK)
  def _(i):
    off = base + i * CHUNK
    pltpu.sync_copy(src_hbm.at[pl.ds(off, CHUNK)], sidx)
    pltpu.sync_copy(dst_hbm.at[pl.ds(off, CHUNK)], didx)
    pltpu.sync_copy(xp_hbm.at[sidx], rows)
    for j in range(CHUNK // L):
      d16 = didx[pl.ds(j * L, L)]
      plsc.addupdate_scatter(cnt, [d16], ones16)
    pltpu.sync_copy(rows, acc.at[didx], add=True)

  plsc.subcore_barrier()
  pltpu.sync_copy(cnt, out_cnt.at[wid])
  for k in range(RPT // CHUNK):
    sl = pl.ds(sid * RPT + k * CHUNK, CHUNK)
    pltpu.sync_copy(acc.at[sl], out_sum.at[cid, sl])


def _sage_agg(src_pa, dst_pa, x_paper):
  return pl.kernel(
      _sage_body,
      out_type=[jax.ShapeDtypeStruct((NC, NROW, D), _f32),
                jax.ShapeDtypeStruct((NW, NROW), _f32)],
      mesh=_mesh,
      compiler_params=pltpu.CompilerParams(needs_layout_passes=False),
      scratch_types=[
          pltpu.VMEM((CHUNK,), _i32),
          pltpu.VMEM((CHUNK,), _i32),
          pltpu.VMEM((CHUNK, D), _f32),
          pltpu.VMEM((NROW,), _f32),
          pltpu.VMEM_SHARED((NROW, D), _f32),
      ],
  )(src_pa, dst_pa, x_paper)


def _p2a_body(p_hbm, a_hbm, out_tab, pbuf, abuf, tab):
  cid = lax.axis_index("c")
  sid = lax.axis_index("s")
  wid = sid * NC + cid
  neg16 = jnp.full((L,), -1, _i32)

  @pl.loop(0, NROW // L)
  def _(i):
    tab[pl.ds(i * L, L)] = neg16

  pbuf[pl.ds(CHUNK, L)] = neg16
  base = wid * TE
  iota = lax.iota(_i32, L)

  @pl.loop(0, NCHUNK)
  def _(i):
    off = base + i * CHUNK
    pltpu.sync_copy(p_hbm.at[pl.ds(off, CHUNK)], pbuf.at[pl.ds(0, CHUNK)])
    pltpu.sync_copy(a_hbm.at[pl.ds(off, CHUNK)], abuf)
    for j in range(CHUNK // L):
      sl = pl.ds(j * L, L)
      p16 = pbuf[sl]
      a16 = abuf[sl]
      dup = p16 != p16
      for s in range(1, L):
        dup = dup | (p16 == pbuf[pl.ds(j * L + s, L)])
      valid = (off + j * L + iota) < NE
      keep = valid & jnp.logical_not(dup)
      plsc.store_scatter(tab, [p16], a16, mask=keep)

  pltpu.sync_copy(tab, out_tab.at[wid])


def _p2a_sc(papers, authors):
  return pl.kernel(
      _p2a_body,
      out_type=[jax.ShapeDtypeStruct((NW, NROW), _i32)],
      mesh=_mesh,
      compiler_params=pltpu.CompilerParams(needs_layout_passes=False),
      scratch_types=[
          pltpu.VMEM((CHUNK + L,), _i32),
          pltpu.VMEM((CHUNK,), _i32),
          pltpu.VMEM((NROW,), _i32),
      ],
  )(papers, authors)[0]


def _gatw_body(src_hbm, pap_hbm, p2a_hbm, as_hbm, ad_hbm, perm_hbm,
               out_wp, out_wn, out_gn, out_dd, out_denp, out_denn,
               sbuf, pbuf, wpb, wnb, gnb, ddb, p2a_v, as_v, ad_v, perm_v,
               denp_v, denn_v):
  cid = lax.axis_index("c")
  sid = lax.axis_index("s")
  wid = sid * NC + cid
  zero16 = jnp.zeros((L,), _f32)

  pltpu.sync_copy(p2a_hbm, p2a_v)
  pltpu.sync_copy(as_hbm, as_v)
  pltpu.sync_copy(ad_hbm, ad_v)
  pltpu.sync_copy(perm_hbm, perm_v)

  @pl.loop(0, NROW // L)
  def _(i):
    denp_v[pl.ds(i * L, L)] = zero16
    denn_v[pl.ds(i * L, L)] = zero16

  base = wid * TE

  @pl.loop(0, NCHUNK)
  def _(i):
    off = base + i * CHUNK
    pltpu.sync_copy(src_hbm.at[pl.ds(off, CHUNK)], sbuf)
    pltpu.sync_copy(pap_hbm.at[pl.ds(off, CHUNK)], pbuf)
    for j in range(CHUNK // L):
      sl = pl.ds(j * L, L)
      s16 = sbuf[sl]
      p16 = pbuf[sl]
      mapped = plsc.load_gather(p2a_v, [p16])
      mask = mapped >= 0
      d0 = jnp.where(mask, mapped, 0)
      dsc = jnp.where(mask, mapped, JROW)

      def edge_w(si, di):
        e = plsc.load_gather(as_v, [si]) + plsc.load_gather(ad_v, [di])
        e = jnp.where(e >= 0.0, e, e * 0.2)
        return jnp.where(mask, jnp.exp(e), 0.0)

      wp = edge_w(s16, d0)
      sgn = plsc.load_gather(perm_v, [s16])
      dgn = plsc.load_gather(perm_v, [d0])
      wn = edge_w(sgn, dgn)
      plsc.addupdate_scatter(denp_v, [dsc], wp, mask=mask)
      plsc.addupdate_scatter(denn_v, [dsc], wn, mask=mask)
      wpb[sl] = wp
      wnb[sl] = wn
      gnb[sl] = sgn
      ddb[sl] = dsc
    pltpu.sync_copy(wpb, out_wp.at[pl.ds(off, CHUNK)])
    pltpu.sync_copy(wnb, out_wn.at[pl.ds(off, CHUNK)])
    pltpu.sync_copy(gnb, out_gn.at[pl.ds(off, CHUNK)])
    pltpu.sync_copy(ddb, out_dd.at[pl.ds(off, CHUNK)])

  pltpu.sync_copy(denp_v, out_denp.at[wid])
  pltpu.sync_copy(denn_v, out_denn.at[wid])


def _gat_weights(src_ap, pap_ap, p2a_pad, a_s, a_d, perm_tab):
  return pl.kernel(
      _gatw_body,
      out_type=[jax.ShapeDtypeStruct((EPAD,), _f32),
                jax.ShapeDtypeStruct((EPAD,), _f32),
                jax.ShapeDtypeStruct((EPAD,), _i32),
                jax.ShapeDtypeStruct((EPAD,), _i32),
                jax.ShapeDtypeStruct((NW, NROW), _f32),
                jax.ShapeDtypeStruct((NW, NROW), _f32)],
      mesh=_mesh,
      compiler_params=pltpu.CompilerParams(needs_layout_passes=False),
      scratch_types=[
          pltpu.VMEM((CHUNK,), _i32),
          pltpu.VMEM((CHUNK,), _i32),
          pltpu.VMEM((CHUNK,), _f32),
          pltpu.VMEM((CHUNK,), _f32),
          pltpu.VMEM((CHUNK,), _i32),
          pltpu.VMEM((CHUNK,), _i32),
          pltpu.VMEM((NROW,), _i32),
          pltpu.VMEM((NA,), _f32),
          pltpu.VMEM((NA,), _f32),
          pltpu.VMEM((NA,), _i32),
          pltpu.VMEM((NROW,), _f32),
          pltpu.VMEM((NROW,), _f32),
      ],
  )(src_ap, pap_ap, p2a_pad, a_s, a_d, perm_tab)


def _gatr_body(gidx_hbm, didx_hbm, w_hbm, h_hbm, out_num,
               gbuf, dbuf, wbuf, rows, acc):
  cid = lax.axis_index("c")
  sid = lax.axis_index("s")
  wid = sid * NC + cid
  zero16 = jnp.zeros((L,), _f32)

  @pl.loop(0, CHUNK)
  def _(r):
    for c in range(D // L):
      rows[r, pl.ds(c * L, L)] = zero16

  for k in range(RPT // CHUNK):
    pltpu.sync_copy(rows, acc.at[pl.ds(sid * RPT + k * CHUNK, CHUNK)])
  plsc.subcore_barrier()

  base = wid * TE

  @pl.loop(0, NCHUNK)
  def _(i):
    off = base + i * CHUNK
    pltpu.sync_copy(gidx_hbm.at[pl.ds(off, CHUNK)], gbuf)
    pltpu.sync_copy(didx_hbm.at[pl.ds(off, CHUNK)], dbuf)
    pltpu.sync_copy(w_hbm.at[pl.ds(off, CHUNK)], wbuf)
    pltpu.sync_copy(h_hbm.at[gbuf], rows)

    @pl.loop(0, CHUNK // L)
    def _(g):
      w16 = wbuf[pl.ds(g * L, L)]
      for kk in range(L):
        wr = w16[kk]
        r = g * L + kk
        for c in range(D // L):
          sl2 = pl.ds(c * L, L)
          rows[r, sl2] = rows[r, sl2] * wr

    pltpu.sync_copy(rows, acc.at[dbuf], add=True)

  plsc.subcore_barrier()
  for k in range(RPT // CHUNK):
    sl = pl.ds(sid * RPT + k * CHUNK, CHUNK)
    pltpu.sync_copy(acc.at[sl], out_num.at[cid, sl])


def _gat_rows(gidx, didx, w, h):
  return pl.kernel(
      _gatr_body,
      out_type=[jax.ShapeDtypeStruct((NC, NROW, D), _f32)],
      mesh=_mesh,
      compiler_params=pltpu.CompilerParams(needs_layout_passes=False),
      scratch_types=[
          pltpu.VMEM((CHUNK,), _i32),
          pltpu.VMEM((CHUNK,), _i32),
          pltpu.VMEM((CHUNK,), _f32),
          pltpu.VMEM((CHUNK, D), _f32),
          pltpu.VMEM_SHARED((NROW, D), _f32),
      ],
  )(gidx, didx, w, h)


def _perm_body(h_hbm, ws_hbm, perm_hbm, out_hp, out_wsp,
               ibuf, rows, wtab, wout):
  cid = lax.axis_index("c")
  sid = lax.axis_index("s")
  wid = sid * NC + cid
  pltpu.sync_copy(ws_hbm, wtab)
  base = wid * PPT

  @pl.loop(0, PPT // PCH)
  def _(k):
    off = base + k * PCH
    pltpu.sync_copy(perm_hbm.at[pl.ds(off, PCH)], ibuf)
    pltpu.sync_copy(h_hbm.at[ibuf], rows)
    pltpu.sync_copy(rows, out_hp.at[pl.ds(off, PCH)])
    for g in range(PCH // L):
      sl = pl.ds(g * L, L)
      wout[sl] = plsc.load_gather(wtab, [ibuf[sl]])
    pltpu.sync_copy(wout, out_wsp.at[pl.ds(off, PCH)])


def _permute(h, ws_flat, perm_pad):
  return pl.kernel(
      _perm_body,
      out_type=[jax.ShapeDtypeStruct((NROW, D), _f32),
                jax.ShapeDtypeStruct((NROW,), _f32)],
      mesh=_mesh,
      compiler_params=pltpu.CompilerParams(needs_layout_passes=False),
      scratch_types=[
          pltpu.VMEM((PCH,), _i32),
          pltpu.VMEM((PCH, D), _f32),
          pltpu.VMEM((NA,), _f32),
          pltpu.VMEM((PCH,), _f32),
      ],
  )(h, ws_flat, perm_pad)


_BLK = 2048
_GRID = NROW // _BLK


def _tc1_body(sum_ref, cnt_ref, pp_ref, xa_ref, wl_ref, bl_ref, wr_ref,
              wg_ref, as_ref, ad_ref, h_ref, asv_ref, adv_ref, ws_ref,
              p2a_ref):
  m = pp_ref[0]
  for t in range(1, NW):
    row = pp_ref[t]
    m = jnp.where(row >= 0, row, m)
  p2a_ref[...] = m[:, None]
  dn = (((1,), (1,)), ((), ()))
  s = sum_ref[0] + sum_ref[1]
  c = jnp.sum(cnt_ref[...], axis=0)
  mean = s / jnp.maximum(c, 1.0)[:, None]
  ha = lax.dot_general(mean, wl_ref[...], dn, preferred_element_type=_f32)
  ha = ha + bl_ref[...]
  ha = ha + lax.dot_general(xa_ref[...], wr_ref[...], dn,
                            preferred_element_type=_f32)
  ha = jnp.where(ha >= 0.0, ha, 0.01 * ha)
  h = lax.dot_general(ha, wg_ref[...], dn, preferred_element_type=_f32)
  h_ref[...] = h
  dv = (((1,), (0,)), ((), ()))
  a_s = lax.dot_general(h, as_ref[...], dv, preferred_element_type=_f32)
  a_d = lax.dot_general(h, ad_ref[...], dv, preferred_element_type=_f32)
  asv_ref[...] = a_s
  adv_ref[...] = a_d
  e = a_s + a_d
  e = jnp.where(e >= 0.0, e, 0.2 * e)
  ws_ref[...] = jnp.exp(e)


def _tc1(sum2, cnt32, p2a_parts, xa_pad, Wl, bl, Wr, Wg, att_s, att_d):
  col = pl.BlockSpec((_BLK, 1), lambda i: (i, 0))
  full = pl.BlockSpec((D, D), lambda i: (0, 0))
  return pl.pallas_call(
      _tc1_body,
      grid=(_GRID,),
      in_specs=[pl.BlockSpec((NC, _BLK, D), lambda i: (0, i, 0)),
                pl.BlockSpec((NW, _BLK), lambda i: (0, i)),
                pl.BlockSpec((NW, _BLK), lambda i: (0, i)),
                pl.BlockSpec((_BLK, D), lambda i: (i, 0)),
                full,
                pl.BlockSpec((1, D), lambda i: (0, 0)),
                full, full,
                pl.BlockSpec((D, 1), lambda i: (0, 0)),
                pl.BlockSpec((D, 1), lambda i: (0, 0))],
      out_specs=[pl.BlockSpec((_BLK, D), lambda i: (i, 0)), col, col, col,
                 col],
      out_shape=[jax.ShapeDtypeStruct((NROW, D), _f32),
                 jax.ShapeDtypeStruct((NROW, 1), _f32),
                 jax.ShapeDtypeStruct((NROW, 1), _f32),
                 jax.ShapeDtypeStruct((NROW, 1), _f32),
                 jax.ShapeDtypeStruct((NROW, 1), _i32)],
  )(sum2, cnt32, p2a_parts, xa_pad, Wl, bl, Wr, Wg, att_s, att_d)


def _tc2_body(np_ref, dp_ref, nn_ref, dn_ref, h_ref, ws_ref, hp_ref, wsp_ref,
              bg_ref, pa_ref, pos_ref, neg_ref, sum_ref):
  i = pl.program_id(0)
  a = pa_ref[0, 0]

  def fin(nref, dref, hv, wv):
    num = nref[0] + nref[1] + wv * hv
    den = jnp.sum(dref[...], axis=0)[:, None] + wv
    o = num / den + bg_ref[...]
    return jnp.where(o >= 0.0, o, a * o)

  pos = fin(np_ref, dp_ref, h_ref[...], ws_ref[...])
  neg = fin(nn_ref, dn_ref, hp_ref[...], wsp_ref[...])
  pos_ref[...] = pos
  neg_ref[...] = neg
  rid = lax.broadcasted_iota(_i32, (_BLK, 1), 0) + i * _BLK
  part = jnp.sum(jnp.where(rid < NA, pos, 0.0), axis=0, keepdims=True)

  @pl.when(i == 0)
  def _():
    sum_ref[...] = jnp.zeros_like(sum_ref)

  sum_ref[...] += part

  @pl.when(i == _GRID - 1)
  def _():
    sum_ref[...] = sum_ref[...] * (1.0 / NA)


def _tc2(nump, denp, numn, denn, h, ws, hp, wsp, bg, pa):
  col = pl.BlockSpec((_BLK, 1), lambda i: (i, 0))
  mat = pl.BlockSpec((_BLK, D), lambda i: (i, 0))
  return pl.pallas_call(
      _tc2_body,
      grid=(_GRID,),
      in_specs=[pl.BlockSpec((NC, _BLK, D), lambda i: (0, i, 0)),
                pl.BlockSpec((NW, _BLK), lambda i: (0, i)),
                pl.BlockSpec((NC, _BLK, D), lambda i: (0, i, 0)),
                pl.BlockSpec((NW, _BLK), lambda i: (0, i)),
                mat, col, mat, col,
                pl.BlockSpec((1, D), lambda i: (0, 0)),
                pl.BlockSpec((1, 1), lambda i: (0, 0))],
      out_specs=[mat, mat, pl.BlockSpec((1, D), lambda i: (0, 0))],
      out_shape=[jax.ShapeDtypeStruct((NROW, D), _f32),
                 jax.ShapeDtypeStruct((NROW, D), _f32),
                 jax.ShapeDtypeStruct((1, D), _f32)],
  )(nump, denp, numn, denn, h, ws, hp, wsp, bg, pa)


@jax.jit
def kernel(x_author, x_paper, W_l_ap, b_l_ap, W_r_ap, W_l_pa, b_l_pa, W_r_pa,
           W_gat, att_src, att_dst, b_gat, prelu_a, edge_index_ap,
           edge_index_pa, perm):
  src_pa = edge_index_pa[0].astype(_i32)
  dst_pa = edge_index_pa[1].astype(_i32)
  src_ap = edge_index_ap[0].astype(_i32)
  pap_ap = edge_index_ap[1].astype(_i32)
  permc = perm.astype(_i32)

  npad = EPAD - NE
  src_pa_p = jnp.concatenate([src_pa, jnp.zeros((npad,), _i32)])
  dst_pa_p = jnp.concatenate([dst_pa, jnp.full((npad,), JROW, _i32)])
  src_ap_p = jnp.concatenate([src_ap, jnp.zeros((npad,), _i32)])
  pap_ap_p = jnp.concatenate([pap_ap, jnp.full((npad,), JROW, _i32)])

  perm_pad = jnp.concatenate([permc, jnp.zeros((NROW - NA,), _i32)])
  xa_pad = jnp.concatenate([x_author, jnp.zeros((NROW - NA, D), _f32)])

  sum2, cnt32 = _sage_agg(src_pa_p, dst_pa_p, x_paper)
  p2a_parts = _p2a_sc(src_pa_p, dst_pa_p)
  h, a_s2, a_d2, ws2, p2a2 = _tc1(sum2, cnt32, p2a_parts, xa_pad, W_l_pa,
                                  b_l_pa.reshape(1, D), W_r_pa, W_gat,
                                  att_src.reshape(D, 1), att_dst.reshape(D, 1))
  p2a_pad = p2a2.reshape(NROW)
  a_s = a_s2[:NA, 0]
  a_d = a_d2[:NA, 0]
  ws_flat = ws2[:NA, 0]
  hp, wsp = _permute(h, ws_flat, perm_pad)
  wp, wn, gn, dd, denp, denn = _gat_weights(src_ap_p, pap_ap_p, p2a_pad,
                                            a_s, a_d, permc)
  nump = _gat_rows(src_ap_p, dd, wp, h)[0]
  numn = _gat_rows(gn, dd, wn, h)[0]
  pos_f, neg_f, summ = _tc2(nump, denp, numn, denn, h, ws2, hp,
                            wsp.reshape(NROW, 1), b_gat.reshape(1, D),
                            prelu_a.reshape(1, 1))
  return pos_f[:NA], neg_f[:NA], summ.reshape(D)


# speedup vs baseline: 4.6404x; 1.1502x over previous
import functools

import jax
import jax.numpy as jnp
from jax import lax
from jax.experimental import pallas as pl
from jax.experimental.pallas import tpu as pltpu
from jax.experimental.pallas import tpu_sc as plsc

NA = 10000
NP = 10000
NE = 320000
D = 128

NC = 2
NS = 16
NW = NC * NS
L = 16

NROW = 10240
JROW = 10000
TE = NROW
EPAD = NW * TE
CHUNK = 128
NCHUNK = TE // CHUNK
RPT = NROW // NS
NCK0 = 105
NCK1 = NCHUNK * 2 - NCK0
TE0 = NCK0 * CHUNK
TE1 = NCK1 * CHUNK
E0 = NS * TE0
PCH = 64
PPT = NROW // NW

_mesh = plsc.VectorSubcoreMesh(
    core_axis_name="c", subcore_axis_name="s", num_cores=NC, num_subcores=NS)

_f32 = jnp.float32
_i32 = jnp.int32


def _sage_body(src_hbm, dst_hbm, xp_hbm, out_sum, out_cnt,
               sidx, didx, rows, cnt, acc):
  cid = lax.axis_index("c")
  sid = lax.axis_index("s")
  wid = sid * NC + cid
  zero16 = jnp.zeros((L,), _f32)

  @pl.loop(0, NROW // L)
  def _(i):
    cnt[pl.ds(i * L, L)] = zero16

  @pl.loop(0, CHUNK)
  def _(r):
    for c in range(D // L):
      rows[r, pl.ds(c * L, L)] = zero16

  for k in range(RPT // CHUNK):
    pltpu.sync_copy(rows, acc.at[pl.ds(sid * RPT + k * CHUNK, CHUNK)])
  plsc.subcore_barrier()

  ones16 = jnp.full((L,), 1.0, _f32)
  base = jnp.where(cid == 0, sid * TE0, E0 + sid * TE1)
  nck = jnp.where(cid == 0, NCK0, NCK1)

  @pl.loop(0, nck)
  def _(i):
    off = base + i * CHUNK
    pltpu.sync_copy(src_hbm.at[pl.ds(off, CHUNK)], sidx)
    pltpu.sync_copy(dst_hbm.at[pl.ds(off, CHUNK)], didx)
    pltpu.sync_copy(xp_hbm.at[sidx], rows)
    for j in range(CHUNK // L):
      d16 = didx[pl.ds(j * L, L)]
      plsc.addupdate_scatter(cnt, [d16], ones16)
    pltpu.sync_copy(rows, acc.at[didx], add=True)

  plsc.subcore_barrier()
  pltpu.sync_copy(cnt, out_cnt.at[wid])
  for k in range(RPT // CHUNK):
    sl = pl.ds(sid * RPT + k * CHUNK, CHUNK)
    pltpu.sync_copy(acc.at[sl], out_sum.at[cid, sl])


def _sage_agg(src_pa, dst_pa, x_paper):
  return pl.kernel(
      _sage_body,
      out_type=[jax.ShapeDtypeStruct((NC, NROW, D), _f32),
                jax.ShapeDtypeStruct((NW, NROW), _f32)],
      mesh=_mesh,
      compiler_params=pltpu.CompilerParams(needs_layout_passes=False),
      scratch_types=[
          pltpu.VMEM((CHUNK,), _i32),
          pltpu.VMEM((CHUNK,), _i32),
          pltpu.VMEM((CHUNK, D), _f32),
          pltpu.VMEM((NROW,), _f32),
          pltpu.VMEM_SHARED((NROW, D), _f32),
      ],
  )(src_pa, dst_pa, x_paper)


def _p2a_body(p_hbm, a_hbm, out_tab, pbuf, abuf, tab):
  cid = lax.axis_index("c")
  sid = lax.axis_index("s")
  wid = sid * NC + cid
  neg16 = jnp.full((L,), -1, _i32)

  @pl.loop(0, NROW // L)
  def _(i):
    tab[pl.ds(i * L, L)] = neg16

  pbuf[pl.ds(CHUNK, L)] = neg16
  base = wid * TE
  iota = lax.iota(_i32, L)

  @pl.loop(0, NCHUNK)
  def _(i):
    off = base + i * CHUNK
    pltpu.sync_copy(p_hbm.at[pl.ds(off, CHUNK)], pbuf.at[pl.ds(0, CHUNK)])
    pltpu.sync_copy(a_hbm.at[pl.ds(off, CHUNK)], abuf)
    for j in range(CHUNK // L):
      sl = pl.ds(j * L, L)
      p16 = pbuf[sl]
      a16 = abuf[sl]
      dup = p16 != p16
      for s in range(1, L):
        dup = dup | (p16 == pbuf[pl.ds(j * L + s, L)])
      valid = (off + j * L + iota) < NE
      keep = valid & jnp.logical_not(dup)
      plsc.store_scatter(tab, [p16], a16, mask=keep)

  pltpu.sync_copy(tab, out_tab.at[wid])


def _p2a_sc(papers, authors):
  return pl.kernel(
      _p2a_body,
      out_type=[jax.ShapeDtypeStruct((NW, NROW), _i32)],
      mesh=_mesh,
      compiler_params=pltpu.CompilerParams(needs_layout_passes=False),
      scratch_types=[
          pltpu.VMEM((CHUNK + L,), _i32),
          pltpu.VMEM((CHUNK,), _i32),
          pltpu.VMEM((NROW,), _i32),
      ],
  )(papers, authors)[0]


def _gatw_body(src_hbm, pap_hbm, p2a_hbm, as_hbm, ad_hbm, perm_hbm,
               out_wp, out_wn, out_gn, out_dd, out_denp, out_denn,
               sbuf, pbuf, wpb, wnb, gnb, ddb, p2a_v, as_v, ad_v, perm_v,
               denp_v, denn_v):
  cid = lax.axis_index("c")
  sid = lax.axis_index("s")
  wid = sid * NC + cid
  zero16 = jnp.zeros((L,), _f32)

  pltpu.sync_copy(p2a_hbm, p2a_v)
  pltpu.sync_copy(as_hbm, as_v)
  pltpu.sync_copy(ad_hbm, ad_v)
  pltpu.sync_copy(perm_hbm, perm_v)

  @pl.loop(0, NROW // L)
  def _(i):
    denp_v[pl.ds(i * L, L)] = zero16
    denn_v[pl.ds(i * L, L)] = zero16

  base = wid * TE

  @pl.loop(0, NCHUNK)
  def _(i):
    off = base + i * CHUNK
    pltpu.sync_copy(src_hbm.at[pl.ds(off, CHUNK)], sbuf)
    pltpu.sync_copy(pap_hbm.at[pl.ds(off, CHUNK)], pbuf)
    for j in range(CHUNK // L):
      sl = pl.ds(j * L, L)
      s16 = sbuf[sl]
      p16 = pbuf[sl]
      mapped = plsc.load_gather(p2a_v, [p16])
      mask = mapped >= 0
      d0 = jnp.where(mask, mapped, 0)
      dsc = jnp.where(mask, mapped, JROW)

      def edge_w(si, di):
        e = plsc.load_gather(as_v, [si]) + plsc.load_gather(ad_v, [di])
        e = jnp.where(e >= 0.0, e, e * 0.2)
        return jnp.where(mask, jnp.exp(e), 0.0)

      wp = edge_w(s16, d0)
      sgn = plsc.load_gather(perm_v, [s16])
      dgn = plsc.load_gather(perm_v, [d0])
      wn = edge_w(sgn, dgn)
      plsc.addupdate_scatter(denp_v, [dsc], wp, mask=mask)
      plsc.addupdate_scatter(denn_v, [dsc], wn, mask=mask)
      wpb[sl] = wp
      wnb[sl] = wn
      gnb[sl] = sgn
      ddb[sl] = dsc
    pltpu.sync_copy(wpb, out_wp.at[pl.ds(off, CHUNK)])
    pltpu.sync_copy(wnb, out_wn.at[pl.ds(off, CHUNK)])
    pltpu.sync_copy(gnb, out_gn.at[pl.ds(off, CHUNK)])
    pltpu.sync_copy(ddb, out_dd.at[pl.ds(off, CHUNK)])

  pltpu.sync_copy(denp_v, out_denp.at[wid])
  pltpu.sync_copy(denn_v, out_denn.at[wid])


def _gat_weights(src_ap, pap_ap, p2a_pad, a_s, a_d, perm_tab):
  return pl.kernel(
      _gatw_body,
      out_type=[jax.ShapeDtypeStruct((EPAD,), _f32),
                jax.ShapeDtypeStruct((EPAD,), _f32),
                jax.ShapeDtypeStruct((EPAD,), _i32),
                jax.ShapeDtypeStruct((EPAD,), _i32),
                jax.ShapeDtypeStruct((NW, NROW), _f32),
                jax.ShapeDtypeStruct((NW, NROW), _f32)],
      mesh=_mesh,
      compiler_params=pltpu.CompilerParams(needs_layout_passes=False),
      scratch_types=[
          pltpu.VMEM((CHUNK,), _i32),
          pltpu.VMEM((CHUNK,), _i32),
          pltpu.VMEM((CHUNK,), _f32),
          pltpu.VMEM((CHUNK,), _f32),
          pltpu.VMEM((CHUNK,), _i32),
          pltpu.VMEM((CHUNK,), _i32),
          pltpu.VMEM((NROW,), _i32),
          pltpu.VMEM((NA,), _f32),
          pltpu.VMEM((NA,), _f32),
          pltpu.VMEM((NA,), _i32),
          pltpu.VMEM((NROW,), _f32),
          pltpu.VMEM((NROW,), _f32),
      ],
  )(src_ap, pap_ap, p2a_pad, a_s, a_d, perm_tab)


def _gatr_body(gidx_hbm, didx_hbm, w_hbm, h_hbm, out_num,
               gbuf, dbuf, wbuf, rows, acc):
  cid = lax.axis_index("c")
  sid = lax.axis_index("s")
  wid = sid * NC + cid
  zero16 = jnp.zeros((L,), _f32)

  @pl.loop(0, CHUNK)
  def _(r):
    for c in range(D // L):
      rows[r, pl.ds(c * L, L)] = zero16

  for k in range(RPT // CHUNK):
    pltpu.sync_copy(rows, acc.at[pl.ds(sid * RPT + k * CHUNK, CHUNK)])
  plsc.subcore_barrier()

  base = jnp.where(cid == 0, sid * TE0, E0 + sid * TE1)
  nck = jnp.where(cid == 0, NCK0, NCK1)

  @pl.loop(0, nck)
  def _(i):
    off = base + i * CHUNK
    pltpu.sync_copy(gidx_hbm.at[pl.ds(off, CHUNK)], gbuf)
    pltpu.sync_copy(didx_hbm.at[pl.ds(off, CHUNK)], dbuf)
    pltpu.sync_copy(w_hbm.at[pl.ds(off, CHUNK)], wbuf)
    pltpu.sync_copy(h_hbm.at[gbuf], rows)

    @pl.loop(0, CHUNK // L)
    def _(g):
      w16 = wbuf[pl.ds(g * L, L)]
      for kk in range(L):
        wr = w16[kk]
        r = g * L + kk
        for c in range(D // L):
          sl2 = pl.ds(c * L, L)
          rows[r, sl2] = rows[r, sl2] * wr

    pltpu.sync_copy(rows, acc.at[dbuf], add=True)

  plsc.subcore_barrier()
  for k in range(RPT // CHUNK):
    sl = pl.ds(sid * RPT + k * CHUNK, CHUNK)
    pltpu.sync_copy(acc.at[sl], out_num.at[cid, sl])


def _gat_rows(gidx, didx, w, h):
  return pl.kernel(
      _gatr_body,
      out_type=[jax.ShapeDtypeStruct((NC, NROW, D), _f32)],
      mesh=_mesh,
      compiler_params=pltpu.CompilerParams(needs_layout_passes=False),
      scratch_types=[
          pltpu.VMEM((CHUNK,), _i32),
          pltpu.VMEM((CHUNK,), _i32),
          pltpu.VMEM((CHUNK,), _f32),
          pltpu.VMEM((CHUNK, D), _f32),
          pltpu.VMEM_SHARED((NROW, D), _f32),
      ],
  )(gidx, didx, w, h)


def _perm_body(h_hbm, ws_hbm, perm_hbm, out_hp, out_wsp,
               ibuf, rows, wtab, wout):
  cid = lax.axis_index("c")
  sid = lax.axis_index("s")
  wid = sid * NC + cid
  pltpu.sync_copy(ws_hbm, wtab)
  base = wid * PPT

  @pl.loop(0, PPT // PCH)
  def _(k):
    off = base + k * PCH
    pltpu.sync_copy(perm_hbm.at[pl.ds(off, PCH)], ibuf)
    pltpu.sync_copy(h_hbm.at[ibuf], rows)
    pltpu.sync_copy(rows, out_hp.at[pl.ds(off, PCH)])
    for g in range(PCH // L):
      sl = pl.ds(g * L, L)
      wout[sl] = plsc.load_gather(wtab, [ibuf[sl]])
    pltpu.sync_copy(wout, out_wsp.at[pl.ds(off, PCH)])


def _permute(h, ws_flat, perm_pad):
  return pl.kernel(
      _perm_body,
      out_type=[jax.ShapeDtypeStruct((NROW, D), _f32),
                jax.ShapeDtypeStruct((NROW,), _f32)],
      mesh=_mesh,
      compiler_params=pltpu.CompilerParams(needs_layout_passes=False),
      scratch_types=[
          pltpu.VMEM((PCH,), _i32),
          pltpu.VMEM((PCH, D), _f32),
          pltpu.VMEM((NA,), _f32),
          pltpu.VMEM((PCH,), _f32),
      ],
  )(h, ws_flat, perm_pad)


_BLK = 2048
_GRID = NROW // _BLK


def _tc1_body(sum_ref, cnt_ref, pp_ref, xa_ref, wl_ref, bl_ref, wr_ref,
              wg_ref, as_ref, ad_ref, h_ref, asv_ref, adv_ref, ws_ref,
              p2a_ref):
  m = pp_ref[0]
  for t in range(1, NW):
    row = pp_ref[t]
    m = jnp.where(row >= 0, row, m)
  p2a_ref[...] = m[:, None]
  dn = (((1,), (1,)), ((), ()))
  s = sum_ref[0] + sum_ref[1]
  c = jnp.sum(cnt_ref[...], axis=0)
  mean = s / jnp.maximum(c, 1.0)[:, None]
  ha = lax.dot_general(mean, wl_ref[...], dn, preferred_element_type=_f32)
  ha = ha + bl_ref[...]
  ha = ha + lax.dot_general(xa_ref[...], wr_ref[...], dn,
                            preferred_element_type=_f32)
  ha = jnp.where(ha >= 0.0, ha, 0.01 * ha)
  h = lax.dot_general(ha, wg_ref[...], dn, preferred_element_type=_f32)
  h_ref[...] = h
  dv = (((1,), (0,)), ((), ()))
  a_s = lax.dot_general(h, as_ref[...], dv, preferred_element_type=_f32)
  a_d = lax.dot_general(h, ad_ref[...], dv, preferred_element_type=_f32)
  asv_ref[...] = a_s
  adv_ref[...] = a_d
  e = a_s + a_d
  e = jnp.where(e >= 0.0, e, 0.2 * e)
  ws_ref[...] = jnp.exp(e)


def _tc1(sum2, cnt32, p2a_parts, xa_pad, Wl, bl, Wr, Wg, att_s, att_d):
  col = pl.BlockSpec((_BLK, 1), lambda i: (i, 0))
  full = pl.BlockSpec((D, D), lambda i: (0, 0))
  return pl.pallas_call(
      _tc1_body,
      grid=(_GRID,),
      in_specs=[pl.BlockSpec((NC, _BLK, D), lambda i: (0, i, 0)),
                pl.BlockSpec((NW, _BLK), lambda i: (0, i)),
                pl.BlockSpec((NW, _BLK), lambda i: (0, i)),
                pl.BlockSpec((_BLK, D), lambda i: (i, 0)),
                full,
                pl.BlockSpec((1, D), lambda i: (0, 0)),
                full, full,
                pl.BlockSpec((D, 1), lambda i: (0, 0)),
                pl.BlockSpec((D, 1), lambda i: (0, 0))],
      out_specs=[pl.BlockSpec((_BLK, D), lambda i: (i, 0)), col, col, col,
                 col],
      out_shape=[jax.ShapeDtypeStruct((NROW, D), _f32),
                 jax.ShapeDtypeStruct((NROW, 1), _f32),
                 jax.ShapeDtypeStruct((NROW, 1), _f32),
                 jax.ShapeDtypeStruct((NROW, 1), _f32),
                 jax.ShapeDtypeStruct((NROW, 1), _i32)],
  )(sum2, cnt32, p2a_parts, xa_pad, Wl, bl, Wr, Wg, att_s, att_d)


def _tc2_body(np_ref, dp_ref, nn_ref, dn_ref, h_ref, ws_ref, hp_ref, wsp_ref,
              bg_ref, pa_ref, pos_ref, neg_ref, sum_ref):
  i = pl.program_id(0)
  a = pa_ref[0, 0]

  def fin(nref, dref, hv, wv):
    num = nref[0] + nref[1] + wv * hv
    den = jnp.sum(dref[...], axis=0)[:, None] + wv
    o = num / den + bg_ref[...]
    return jnp.where(o >= 0.0, o, a * o)

  pos = fin(np_ref, dp_ref, h_ref[...], ws_ref[...])
  neg = fin(nn_ref, dn_ref, hp_ref[...], wsp_ref[...])
  pos_ref[...] = pos
  neg_ref[...] = neg
  rid = lax.broadcasted_iota(_i32, (_BLK, 1), 0) + i * _BLK
  part = jnp.sum(jnp.where(rid < NA, pos, 0.0), axis=0, keepdims=True)

  @pl.when(i == 0)
  def _():
    sum_ref[...] = jnp.zeros_like(sum_ref)

  sum_ref[...] += part

  @pl.when(i == _GRID - 1)
  def _():
    sum_ref[...] = sum_ref[...] * (1.0 / NA)


def _tc2(nump, denp, numn, denn, h, ws, hp, wsp, bg, pa):
  col = pl.BlockSpec((_BLK, 1), lambda i: (i, 0))
  mat = pl.BlockSpec((_BLK, D), lambda i: (i, 0))
  return pl.pallas_call(
      _tc2_body,
      grid=(_GRID,),
      in_specs=[pl.BlockSpec((NC, _BLK, D), lambda i: (0, i, 0)),
                pl.BlockSpec((NW, _BLK), lambda i: (0, i)),
                pl.BlockSpec((NC, _BLK, D), lambda i: (0, i, 0)),
                pl.BlockSpec((NW, _BLK), lambda i: (0, i)),
                mat, col, mat, col,
                pl.BlockSpec((1, D), lambda i: (0, 0)),
                pl.BlockSpec((1, 1), lambda i: (0, 0))],
      out_specs=[mat, mat, pl.BlockSpec((1, D), lambda i: (0, 0))],
      out_shape=[jax.ShapeDtypeStruct((NROW, D), _f32),
                 jax.ShapeDtypeStruct((NROW, D), _f32),
                 jax.ShapeDtypeStruct((1, D), _f32)],
  )(nump, denp, numn, denn, h, ws, hp, wsp, bg, pa)


@jax.jit
def kernel(x_author, x_paper, W_l_ap, b_l_ap, W_r_ap, W_l_pa, b_l_pa, W_r_pa,
           W_gat, att_src, att_dst, b_gat, prelu_a, edge_index_ap,
           edge_index_pa, perm):
  src_pa = edge_index_pa[0].astype(_i32)
  dst_pa = edge_index_pa[1].astype(_i32)
  src_ap = edge_index_ap[0].astype(_i32)
  pap_ap = edge_index_ap[1].astype(_i32)
  permc = perm.astype(_i32)

  npad = EPAD - NE
  src_pa_p = jnp.concatenate([src_pa, jnp.zeros((npad,), _i32)])
  dst_pa_p = jnp.concatenate([dst_pa, jnp.full((npad,), JROW, _i32)])
  src_ap_p = jnp.concatenate([src_ap, jnp.zeros((npad,), _i32)])
  pap_ap_p = jnp.concatenate([pap_ap, jnp.full((npad,), JROW, _i32)])

  perm_pad = jnp.concatenate([permc, jnp.zeros((NROW - NA,), _i32)])
  xa_pad = jnp.concatenate([x_author, jnp.zeros((NROW - NA, D), _f32)])

  sum2, cnt32 = _sage_agg(src_pa_p, dst_pa_p, x_paper)
  p2a_parts = _p2a_sc(src_pa_p, dst_pa_p)
  h, a_s2, a_d2, ws2, p2a2 = _tc1(sum2, cnt32, p2a_parts, xa_pad, W_l_pa,
                                  b_l_pa.reshape(1, D), W_r_pa, W_gat,
                                  att_src.reshape(D, 1), att_dst.reshape(D, 1))
  p2a_pad = p2a2.reshape(NROW)
  a_s = a_s2[:NA, 0]
  a_d = a_d2[:NA, 0]
  ws_flat = ws2[:NA, 0]
  hp, wsp = _permute(h, ws_flat, perm_pad)
  wp, wn, gn, dd, denp, denn = _gat_weights(src_ap_p, pap_ap_p, p2a_pad,
                                            a_s, a_d, permc)
  nump = _gat_rows(src_ap_p, dd, wp, h)[0]
  numn = _gat_rows(gn, dd, wn, h)[0]
  pos_f, neg_f, summ = _tc2(nump, denp, numn, denn, h, ws2, hp,
                            wsp.reshape(NROW, 1), b_gat.reshape(1, D),
                            prelu_a.reshape(1, 1))
  return pos_f[:NA], neg_f[:NA], summ.reshape(D)
